# baseline (speedup 1.0000x reference)
import functools

import jax
import jax.numpy as jnp
from jax import lax
from jax.experimental import pallas as pl
from jax.experimental.pallas import tpu as pltpu
from jax.experimental.pallas import tpu_sc as plsc

N_NODES = 10000
D_FEAT = 128
N_EDGES = 320000

NC = 2
NS = 16
NW = NC * NS
LANES = 16

EPW = N_EDGES // NW
CHUNK = 16
NCHUNK = EPW // CHUNK
GROUPS = CHUNK // LANES
NBUF = 5


def _sc_body(table, nids32, score_out, srch_out, tgth_out,
             idx_all_s, idx_all_t, score_all, table_sh,
             rows_s0, rows_t0, rows_s1, rows_t1,
             rows_s2, rows_t2, rows_s3, rows_t3, rows_s4, rows_t4, fold_v,
             g0, g1, g2, g3, g4, w0, w1, w2, w3, w4):
  sid = lax.axis_index("s")
  wid = sid * NC + lax.axis_index("c")
  base = wid * EPW

  stripe = 632

  @pl.when(sid < NS - 1)
  def _():
    roff = pl.multiple_of(sid * stripe, 8)
    pltpu.sync_copy(table.at[pl.ds(roff, stripe)],
                    table_sh.at[pl.ds(roff, stripe)])

  @pl.when(sid == NS - 1)
  def _():
    tail = N_NODES - (NS - 1) * stripe
    pltpu.sync_copy(table.at[pl.ds((NS - 1) * stripe, tail)],
                    table_sh.at[pl.ds((NS - 1) * stripe, tail)])

  pltpu.sync_copy(nids32.at[pl.ds(base, EPW)], idx_all_s)
  pltpu.sync_copy(nids32.at[pl.ds(N_EDGES + base, EPW)], idx_all_t)
  plsc.subcore_barrier()

  rows_s = (rows_s0, rows_s1, rows_s2, rows_s3, rows_s4)
  rows_t = (rows_t0, rows_t1, rows_t2, rows_t3, rows_t4)
  gsem = (g0, g1, g2, g3, g4)
  wsem = (w0, w1, w2, w3, w4)

  lane = lax.iota(jnp.int32, LANES)
  fold_v[pl.ds(LANES, LANES)] = jnp.zeros((LANES,), jnp.float32)

  def fire(c, s):
    ioff = pl.multiple_of(c * CHUNK, CHUNK)
    pltpu.async_copy(table_sh.at[idx_all_s.at[pl.ds(ioff, CHUNK)]],
                     rows_s[s], gsem[s])
    pltpu.async_copy(table_sh.at[idx_all_t.at[pl.ds(ioff, CHUNK)]],
                     rows_t[s], gsem[s])

  def drain_gather(s):
    pltpu.make_async_copy(table.at[pl.ds(0, CHUNK)], rows_s[s], gsem[s]).wait()
    pltpu.make_async_copy(table.at[pl.ds(0, CHUNK)], rows_t[s], gsem[s]).wait()

  def drain_wb(s):
    pltpu.make_async_copy(rows_s[s], srch_out.at[pl.ds(0, CHUNK)],
                          wsem[s]).wait()
    pltpu.make_async_copy(rows_t[s], tgth_out.at[pl.ds(0, CHUNK)],
                          wsem[s]).wait()

  def compute(c, s):
    rs, rt = rows_s[s], rows_t[s]
    svec = jnp.zeros((LANES,), jnp.float32)
    for j in range(LANES):
      acc = jnp.zeros((LANES,), jnp.float32)
      for k in range(D_FEAT // LANES):
        a = rs[j, pl.ds(k * LANES, LANES)]
        b = rt[j, pl.ds(k * LANES, LANES)]
        acc = acc + a * b
      fold_v[pl.ds(0, LANES)] = acc
      acc = acc + fold_v[pl.ds(LANES // 2, LANES)]
      tot = acc[0]
      for l in range(1, LANES // 2):
        tot = tot + acc[l]
      svec = jnp.where(lane == j, tot, svec)
    score_all[pl.ds(c * CHUNK, LANES)] = svec

  def issue_wb(c, s):
    off = pl.multiple_of(base + c * CHUNK, 8)
    pltpu.async_copy(rows_s[s], srch_out.at[pl.ds(off, CHUNK)], wsem[s])
    pltpu.async_copy(rows_t[s], tgth_out.at[pl.ds(off, CHUNK)], wsem[s])

  fire(0, 0)
  fire(1, 1)

  def body(p, carry):
    for u in range(NBUF):
      c = p * NBUF + u
      s = u
      s2 = (u + 2) % NBUF
      drain_gather(s)
      compute(c, s)
      issue_wb(c, s)
      if u < 3:
        @pl.when(p >= 1)
        def _():
          drain_wb(s2)
      else:
        drain_wb(s2)
      if u < 3:
        fire(c + 2, s2)
      else:
        @pl.when(c + 2 <= NCHUNK - 1)
        def _():
          fire(c + 2, s2)
    return carry

  lax.fori_loop(0, NCHUNK // NBUF, body, 0)

  drain_wb(2)
  drain_wb(3)
  drain_wb(4)

  pltpu.sync_copy(score_all, score_out.at[pl.ds(base, EPW)])


@jax.jit
def kernel(node_embeddings, node_nids):
  nids32 = node_nids.astype(jnp.int32).reshape(-1)

  mesh = plsc.VectorSubcoreMesh(core_axis_name="c", subcore_axis_name="s")
  out_type = (
      jax.ShapeDtypeStruct((N_EDGES,), jnp.float32),
      jax.ShapeDtypeStruct((N_EDGES, D_FEAT), jnp.float32),
      jax.ShapeDtypeStruct((N_EDGES, D_FEAT), jnp.float32),
  )
  scratch = [
      pltpu.VMEM((EPW,), jnp.int32),
      pltpu.VMEM((EPW,), jnp.int32),
      pltpu.VMEM((EPW,), jnp.float32),
      pltpu.VMEM_SHARED((N_NODES, D_FEAT), jnp.float32),
  ] + [pltpu.VMEM((CHUNK, D_FEAT), jnp.float32) for _ in range(2 * NBUF)] + [
      pltpu.VMEM((2 * LANES,), jnp.float32),
  ] + [
      pltpu.SemaphoreType.DMA for _ in range(2 * NBUF)
  ]
  score, src_h, tgt_h = pl.kernel(
      _sc_body,
      out_type=out_type,
      mesh=mesh,
      scratch_types=scratch,
  )(node_embeddings, nids32)
  return (score, src_h, tgt_h)


# speedup vs baseline: 1.5550x; 1.5550x over previous
import functools

import jax
import jax.numpy as jnp
from jax import lax
from jax.experimental import pallas as pl
from jax.experimental.pallas import tpu as pltpu
from jax.experimental.pallas import tpu_sc as plsc

N_NODES = 10000
D_FEAT = 128
N_EDGES = 320000

NC = 2
NS = 16
NW = NC * NS
LANES = 16

EPW = N_EDGES // NW
CHUNK = 16
NCHUNK = EPW // CHUNK
GROUPS = CHUNK // LANES
NBUF = 4


def _sc_body(table, nids32, score_out, srch_out, tgth_out,
             idx_all_s, idx_all_t, score_all, table_sh,
             rows_s0, rows_t0, rows_s1, rows_t1,
             rows_s2, rows_t2, rows_s3, rows_t3, fold_v,
             g0, g1, g2, g3, w0, w1, w2, w3):
  sid = lax.axis_index("s")
  wid = sid * NC + lax.axis_index("c")
  base = wid * EPW

  stripe = 632

  @pl.when(sid < NS - 1)
  def _():
    roff = pl.multiple_of(sid * stripe, 8)
    pltpu.sync_copy(table.at[pl.ds(roff, stripe)],
                    table_sh.at[pl.ds(roff, stripe)])

  @pl.when(sid == NS - 1)
  def _():
    tail = N_NODES - (NS - 1) * stripe
    pltpu.sync_copy(table.at[pl.ds((NS - 1) * stripe, tail)],
                    table_sh.at[pl.ds((NS - 1) * stripe, tail)])

  pltpu.sync_copy(nids32.at[pl.ds(base, EPW)], idx_all_s)
  pltpu.sync_copy(nids32.at[pl.ds(N_EDGES + base, EPW)], idx_all_t)
  plsc.subcore_barrier()

  rows_s = (rows_s0, rows_s1, rows_s2, rows_s3)
  rows_t = (rows_t0, rows_t1, rows_t2, rows_t3)
  gsem = (g0, g1, g2, g3)
  wsem = (w0, w1, w2, w3)

  lane = lax.iota(jnp.int32, LANES)
  fold_v[pl.ds(LANES, LANES)] = jnp.zeros((LANES,), jnp.float32)

  def fire(c, s):
    ioff = pl.multiple_of(c * CHUNK, CHUNK)
    pltpu.async_copy(table_sh.at[idx_all_s.at[pl.ds(ioff, CHUNK)]],
                     rows_s[s], gsem[s])
    pltpu.async_copy(table_sh.at[idx_all_t.at[pl.ds(ioff, CHUNK)]],
                     rows_t[s], gsem[s])

  def drain_gather(s):
    pltpu.make_async_copy(table.at[pl.ds(0, CHUNK)], rows_s[s], gsem[s]).wait()
    pltpu.make_async_copy(table.at[pl.ds(0, CHUNK)], rows_t[s], gsem[s]).wait()

  def drain_wb(s):
    pltpu.make_async_copy(rows_s[s], srch_out.at[pl.ds(0, CHUNK)],
                          wsem[s]).wait()
    pltpu.make_async_copy(rows_t[s], tgth_out.at[pl.ds(0, CHUNK)],
                          wsem[s]).wait()

  def compute(c, s):
    rs, rt = rows_s[s], rows_t[s]
    svec = jnp.zeros((LANES,), jnp.float32)
    for j in range(LANES):
      acc = jnp.zeros((LANES,), jnp.float32)
      for k in range(D_FEAT // LANES):
        a = rs[j, pl.ds(k * LANES, LANES)]
        b = rt[j, pl.ds(k * LANES, LANES)]
        acc = acc + a * b
      fold_v[pl.ds(0, LANES)] = acc
      acc = acc + fold_v[pl.ds(LANES // 2, LANES)]
      tot = acc[0]
      for l in range(1, LANES // 2):
        tot = tot + acc[l]
      svec = jnp.where(lane == j, tot, svec)
    score_all[pl.ds(c * CHUNK, LANES)] = svec

  def issue_wb(c, s):
    off = pl.multiple_of(base + c * CHUNK, 8)
    pltpu.async_copy(rows_s[s], srch_out.at[pl.ds(off, CHUNK)], wsem[s])
    pltpu.async_copy(rows_t[s], tgth_out.at[pl.ds(off, CHUNK)], wsem[s])

  fire(0, 0)
  fire(1, 1)

  def body(p, carry):
    for u in range(NBUF):
      c = p * NBUF + u
      s = u
      s2 = (u + 2) % NBUF
      drain_gather(s)
      compute(c, s)
      issue_wb(c, s)
      if u < 2:
        @pl.when(p >= 1)
        def _():
          drain_wb(s2)
      else:
        drain_wb(s2)
      if u < NBUF - 1:
        fire(c + 2, s2)
      else:
        @pl.when(c + 2 <= NCHUNK - 1)
        def _():
          fire(c + 2, s2)
    return carry

  lax.fori_loop(0, NCHUNK // NBUF, body, 0)

  drain_gather(0)
  compute(NCHUNK - 1, 0)
  issue_wb(NCHUNK - 1, 0)
  drain_wb(2)
  drain_wb(3)
  drain_wb(0)

  pltpu.sync_copy(score_all, score_out.at[pl.ds(base, EPW)])


@jax.jit
def kernel(node_embeddings, node_nids):
  nids32 = node_nids.astype(jnp.int32).reshape(-1)

  mesh = plsc.VectorSubcoreMesh(core_axis_name="c", subcore_axis_name="s")
  out_type = (
      jax.ShapeDtypeStruct((N_EDGES,), jnp.float32),
      jax.ShapeDtypeStruct((N_EDGES, D_FEAT), jnp.float32),
      jax.ShapeDtypeStruct((N_EDGES, D_FEAT), jnp.float32),
  )
  scratch = [
      pltpu.VMEM((EPW,), jnp.int32),
      pltpu.VMEM((EPW,), jnp.int32),
      pltpu.VMEM((EPW,), jnp.float32),
      pltpu.VMEM_SHARED((N_NODES, D_FEAT), jnp.float32),
  ] + [pltpu.VMEM((CHUNK, D_FEAT), jnp.float32) for _ in range(2 * NBUF)] + [
      pltpu.VMEM((2 * LANES,), jnp.float32),
  ] + [
      pltpu.SemaphoreType.DMA for _ in range(2 * NBUF)
  ]
  score, src_h, tgt_h = pl.kernel(
      _sc_body,
      out_type=out_type,
      mesh=mesh,
      scratch_types=scratch,
  )(node_embeddings, nids32)
  return (score, src_h, tgt_h)


# speedup vs baseline: 1.5614x; 1.0041x over previous
import jax
import jax.numpy as jnp
from jax import lax
from jax.experimental import pallas as pl
from jax.experimental.pallas import tpu as pltpu
from jax.experimental.pallas import tpu_sc as plsc

N_NODES = 10000
D_FEAT = 128
N_EDGES = 320000

NC = 2
NS = 16
NW = NC * NS
LANES = 16

EPW = N_EDGES // NW
CHUNK = 16
NCHUNK = EPW // CHUNK
GROUPS = CHUNK // LANES
NBUF = 4


def _sc_body(table, nids32, score_out, srch_out, tgth_out,
             idx_all_s, idx_all_t, score_all, table_sh,
             rows_s0, rows_t0, rows_s1, rows_t1,
             rows_s2, rows_t2, rows_s3, rows_t3, fold_v,
             g0, g1, g2, g3, w0, w1, w2, w3):
  sid = lax.axis_index("s")
  wid = sid * NC + lax.axis_index("c")
  base = wid * EPW

  stripe = 632

  @pl.when(sid < NS - 1)
  def _():
    roff = pl.multiple_of(sid * stripe, 8)
    pltpu.sync_copy(table.at[pl.ds(roff, stripe)],
                    table_sh.at[pl.ds(roff, stripe)])

  @pl.when(sid == NS - 1)
  def _():
    tail = N_NODES - (NS - 1) * stripe
    pltpu.sync_copy(table.at[pl.ds((NS - 1) * stripe, tail)],
                    table_sh.at[pl.ds((NS - 1) * stripe, tail)])

  pltpu.sync_copy(nids32.at[pl.ds(base, EPW)], idx_all_s)
  pltpu.sync_copy(nids32.at[pl.ds(N_EDGES + base, EPW)], idx_all_t)
  plsc.subcore_barrier()

  rows_s = (rows_s0, rows_s1, rows_s2, rows_s3)
  rows_t = (rows_t0, rows_t1, rows_t2, rows_t3)
  gsem = (g0, g1, g2, g3)
  wsem = (w0, w1, w2, w3)

  lane = lax.iota(jnp.int32, LANES)
  fold_v[pl.ds(LANES, LANES)] = jnp.zeros((LANES,), jnp.float32)

  def fire(c, s):
    ioff = pl.multiple_of(c * CHUNK, CHUNK)
    pltpu.async_copy(table_sh.at[idx_all_s.at[pl.ds(ioff, CHUNK)]],
                     rows_s[s], gsem[s])
    pltpu.async_copy(table_sh.at[idx_all_t.at[pl.ds(ioff, CHUNK)]],
                     rows_t[s], gsem[s])

  def drain_gather(s):
    pltpu.make_async_copy(table.at[pl.ds(0, CHUNK)], rows_s[s], gsem[s]).wait()
    pltpu.make_async_copy(table.at[pl.ds(0, CHUNK)], rows_t[s], gsem[s]).wait()

  def drain_wb(s):
    pltpu.make_async_copy(rows_s[s], srch_out.at[pl.ds(0, CHUNK)],
                          wsem[s]).wait()
    pltpu.make_async_copy(rows_t[s], tgth_out.at[pl.ds(0, CHUNK)],
                          wsem[s]).wait()

  def compute(c, s):
    rs, rt = rows_s[s], rows_t[s]
    svec = jnp.zeros((LANES,), jnp.float32)
    for j in range(LANES):
      acc = jnp.zeros((LANES,), jnp.float32)
      for k in range(D_FEAT // LANES):
        a = rs[j, pl.ds(k * LANES, LANES)]
        b = rt[j, pl.ds(k * LANES, LANES)]
        acc = acc + a * b
      fold_v[pl.ds(0, LANES)] = acc
      acc = acc + fold_v[pl.ds(LANES // 2, LANES)]
      tot = acc[0]
      for l in range(1, LANES // 2):
        tot = tot + acc[l]
      svec = jnp.where(lane == j, tot, svec)
    score_all[pl.ds(c * CHUNK, LANES)] = svec

  def issue_wb(c, s):
    off = pl.multiple_of(base + c * CHUNK, 8)
    pltpu.async_copy(rows_s[s], srch_out.at[pl.ds(off, CHUNK)], wsem[s])
    pltpu.async_copy(rows_t[s], tgth_out.at[pl.ds(off, CHUNK)], wsem[s])

  fire(0, 0)
  fire(1, 1)

  def body(p, carry):
    for u in range(NBUF):
      c = p * NBUF + u
      s = u
      s2 = (u + 2) % NBUF
      drain_gather(s)
      compute(c, s)
      issue_wb(c, s)
      if u < 2:
        @pl.when(p >= 1)
        def _():
          drain_wb(s2)
      else:
        drain_wb(s2)
      if u < NBUF - 1:
        fire(c + 2, s2)
      else:
        @pl.when(c + 2 <= NCHUNK - 1)
        def _():
          fire(c + 2, s2)
    return carry

  lax.fori_loop(0, NCHUNK // NBUF, body, 0)

  drain_gather(0)
  compute(NCHUNK - 1, 0)
  issue_wb(NCHUNK - 1, 0)
  drain_wb(2)
  drain_wb(3)
  drain_wb(0)

  pltpu.sync_copy(score_all, score_out.at[pl.ds(base, EPW)])


@jax.jit
def kernel(node_embeddings, node_nids):
  nids32 = node_nids.astype(jnp.int32).reshape(-1)

  mesh = plsc.VectorSubcoreMesh(core_axis_name="c", subcore_axis_name="s")
  out_type = (
      jax.ShapeDtypeStruct((N_EDGES,), jnp.float32),
      jax.ShapeDtypeStruct((N_EDGES, D_FEAT), jnp.float32),
      jax.ShapeDtypeStruct((N_EDGES, D_FEAT), jnp.float32),
  )
  scratch = [
      pltpu.VMEM((EPW,), jnp.int32),
      pltpu.VMEM((EPW,), jnp.int32),
      pltpu.VMEM((EPW,), jnp.float32),
      pltpu.VMEM_SHARED((N_NODES, D_FEAT), jnp.float32),
  ] + [pltpu.VMEM((CHUNK, D_FEAT), jnp.float32) for _ in range(2 * NBUF)] + [
      pltpu.VMEM((2 * LANES,), jnp.float32),
  ] + [
      pltpu.SemaphoreType.DMA for _ in range(2 * NBUF)
  ]
  score, src_h, tgt_h = pl.kernel(
      _sc_body,
      out_type=out_type,
      mesh=mesh,
      scratch_types=scratch,
  )(node_embeddings, nids32)
  return (score, src_h, tgt_h)
